# baseline (speedup 1.0000x reference)
import functools

import jax
import jax.numpy as jnp
from jax import lax
from jax.experimental import pallas as pl
from jax.experimental.pallas import tpu as pltpu
from jax.experimental.pallas import tpu_sc as plsc

_SQRT_HALF = 0.7071067811865476

_NC, _NS, _L = 2, 16, 16
_NW = _NC * _NS


def _gelu(x):
    return 0.5 * x * (1.0 + jax.lax.erf(x * _SQRT_HALF))


def _dot(a, b):
    return jnp.dot(a, b, preferred_element_type=jnp.float32)


def _bdot(a, b):
    return jnp.dot(a.astype(jnp.bfloat16), b, preferred_element_type=jnp.float32)


def _stage_a(x_ref, Wte_ref, bte_ref, l2e_ref, cle_ref, Wg_ref, bg_ref,
             xh_ref, ltl2_ref, ltcl_ref):
    xh = _gelu(_dot(x_ref[...], Wte_ref[...]) + bte_ref[...])
    xh_ref[...] = xh.astype(jnp.bfloat16)
    gl2 = _dot(xh + l2e_ref[...], Wg_ref[...]) + bg_ref[...]
    gcl = _dot(xh + cle_ref[...], Wg_ref[...]) + bg_ref[...]
    ltl2_ref[...] = gl2.T
    ltcl_ref[...] = gcl.T


def _route_chunk(lt_v, w_v, o, E):
    _BIG = 3.0e38
    vs = [lt_v[e, pl.ds(o, _L)] for e in range(E)]
    m = vs[0]
    for v in vs[1:]:
        m = jnp.maximum(m, v)
    exs = [jnp.exp(v - m) for v in vs]
    z = exs[0]
    for ex in exs[1:]:
        z = z + ex
    zi = 1.0 / z
    s1 = []
    free = None
    for v in vs:
        s = jnp.where(v == m, 1.0 if free is None else free, 0.0)
        free = (1.0 - s) if free is None else (free - s)
        s1.append(s)
    vs2 = [v - s * _BIG for v, s in zip(vs, s1)]
    m2 = vs2[0]
    for v in vs2[1:]:
        m2 = jnp.maximum(m2, v)
    s2 = []
    free2 = None
    for v in vs2:
        s = jnp.where(v == m2, 1.0 if free2 is None else free2, 0.0)
        free2 = (1.0 - s) if free2 is None else (free2 - s)
        s2.append(s)
    for e in range(E):
        w_v[e, pl.ds(o, _L)] = (s1[e] + s2[e]) * (exs[e] * zi)


def _make_router(E, n):
    CH = n // _NW
    mesh = plsc.VectorSubcoreMesh(core_axis_name="c", subcore_axis_name="s")

    @functools.partial(
        pl.kernel,
        out_type=[jax.ShapeDtypeStruct((E, n), jnp.float32),
                  jax.ShapeDtypeStruct((E, n), jnp.float32)],
        mesh=mesh,
        scratch_types=[
            pltpu.VMEM((E, CH), jnp.float32),
            pltpu.VMEM((E, CH), jnp.float32),
            pltpu.VMEM((E, CH), jnp.float32),
            pltpu.VMEM((E, CH), jnp.float32),
        ],
    )
    def _router(ltl2_hbm, ltcl_hbm, wl2_hbm, wcl_hbm, l2_v, cl_v, w1_v, w2_v):
        wid = lax.axis_index("s") * _NC + lax.axis_index("c")
        base = wid * CH
        pltpu.sync_copy(ltl2_hbm.at[:, pl.ds(base, CH)], l2_v)
        pltpu.sync_copy(ltcl_hbm.at[:, pl.ds(base, CH)], cl_v)

        def body(j, carry):
            o = j * _L
            _route_chunk(l2_v, w1_v, o, E)
            _route_chunk(cl_v, w2_v, o, E)
            return carry

        lax.fori_loop(0, CH // _L, body, 0)
        pltpu.sync_copy(w1_v, wl2_hbm.at[:, pl.ds(base, CH)])
        pltpu.sync_copy(w2_v, wcl_hbm.at[:, pl.ds(base, CH)])

    return _router


def _stage_c_core(xh_ref, wl2t_ref, wclt_ref, l2e_ref, cle_ref, W1c_ref,
                  b1f_ref, W2s_ref, b2_ref, rep_ref, lnl2g_ref, lnl2b_ref,
                  lncg_ref, lncb_ref, Wt2i_ref, bt2i_ref, Wcl_ref, bcl_ref,
                  l2r_ref, clr_ref):
    xh = xh_ref[...].astype(jnp.float32)

    def branch(emb_ref, wt_ref, g_ref, b_ref):
        inp = xh + emb_ref[...]
        w = wt_ref[...].T
        zpre = _bdot(inp, W1c_ref[...]) + b1f_ref[...]
        h = _gelu(zpre.astype(jnp.bfloat16))
        wrep = _bdot(w, rep_ref[...]).astype(jnp.bfloat16)
        moe = _dot(h * wrep, W2s_ref[...]) + _dot(w, b2_ref[...])
        mu = jnp.mean(moe, axis=-1, keepdims=True)
        var = jnp.mean((moe - mu) ** 2, axis=-1, keepdims=True)
        ln = g_ref[...] * (moe - mu) * jax.lax.rsqrt(var + 1e-5) + b_ref[...]
        return _gelu(ln) + inp

    l2o = branch(l2e_ref, wl2t_ref, lnl2g_ref, lnl2b_ref)
    clo = branch(cle_ref, wclt_ref, lncg_ref, lncb_ref)
    l2r_ref[...] = _bdot(l2o, Wt2i_ref[...]) + bt2i_ref[...]
    clr_ref[...] = _bdot(clo, Wcl_ref[...]) + bcl_ref[...]


def _stage_c_alias(*refs):
    _stage_c_core(*refs[:18], *refs[20:])


def kernel(x, Wte, bte, l2_emb, cl_emb, Wg, bg, W1, b1, W2, b2,
           ln_l2_g, ln_l2_b, ln_cl_g, ln_cl_b, Wt2i, bt2i, Wcl, bcl):
    B, S, TD = x.shape
    H = Wte.shape[1]
    E = Wg.shape[1]
    N = B * S
    T = min(1024, N)
    NCHUNK = 2 if (N // T) % 2 == 0 else 1
    NH = N // NCHUNK
    CG = NH // T
    xf = x.reshape(N, TD)
    W1c = W1.transpose(1, 0, 2).reshape(H, E * H).astype(jnp.bfloat16)
    b1f = b1.reshape(1, E * H)
    W2s = W2.reshape(E * H, H).astype(jnp.bfloat16)
    Wt2i = Wt2i.astype(jnp.bfloat16)
    Wcl = Wcl.astype(jnp.bfloat16)
    rep = jnp.repeat(jnp.eye(E, dtype=jnp.bfloat16), H, axis=1)

    row = lambda v: v.reshape(1, -1)
    full = lambda shape: pl.BlockSpec(shape, lambda i: (0, 0))

    router = _make_router(E, NH)
    chunks = []
    for c in range(NCHUNK):
        xh_c, ltl2_c, ltcl_c = pl.pallas_call(
            _stage_a,
            grid=(CG,),
            in_specs=[
                pl.BlockSpec((T, TD), lambda i, c=c: (i + c * CG, 0)),
                full((TD, H)), full((1, H)), full((1, H)), full((1, H)),
                full((H, E)), full((1, E)),
            ],
            out_specs=[
                pl.BlockSpec((T, H), lambda i: (i, 0)),
                pl.BlockSpec((E, T), lambda i: (0, i)),
                pl.BlockSpec((E, T), lambda i: (0, i)),
            ],
            out_shape=[
                jax.ShapeDtypeStruct((NH, H), jnp.bfloat16),
                jax.ShapeDtypeStruct((E, NH), jnp.float32),
                jax.ShapeDtypeStruct((E, NH), jnp.float32),
            ],
        )(xf, Wte, row(bte), row(l2_emb), row(cl_emb), Wg, row(bg))
        chunks.append((xh_c, ltl2_c, ltcl_c))

    routed = [router(lt2, ltc) for (_, lt2, ltc) in chunks]

    weight_args = (row(l2_emb), row(cl_emb), W1c, b1f, W2s, b2, rep,
                   row(ln_l2_g), row(ln_l2_b), row(ln_cl_g), row(ln_cl_b),
                   Wt2i, row(bt2i), Wcl, row(bcl))
    weight_specs = [
        full((1, H)), full((1, H)),
        full((H, E * H)), full((1, E * H)),
        full((E * H, H)), full((E, H)), full((E, E * H)),
        full((1, H)), full((1, H)), full((1, H)), full((1, H)),
        full((H, TD)), full((1, TD)),
        full((H, H)), full((1, H)),
    ]
    out_shape = [
        jax.ShapeDtypeStruct((N, TD), jnp.float32),
        jax.ShapeDtypeStruct((N, H), jnp.float32),
    ]

    l2r = clr = None
    for c in range(NCHUNK):
        xh_c, _, _ = chunks[c]
        wl2_c, wcl_c = routed[c]
        data_specs = [
            pl.BlockSpec((T, H), lambda i: (i, 0)),
            pl.BlockSpec((E, T), lambda i: (0, i)),
            pl.BlockSpec((E, T), lambda i: (0, i)),
        ]
        out_specs = [
            pl.BlockSpec((T, TD), lambda i, c=c: (i + c * CG, 0)),
            pl.BlockSpec((T, H), lambda i, c=c: (i + c * CG, 0)),
        ]
        if c == 0:
            l2r, clr = pl.pallas_call(
                _stage_c_core,
                grid=(CG,),
                in_specs=data_specs + weight_specs,
                out_specs=out_specs,
                out_shape=out_shape,
            )(xh_c, wl2_c, wcl_c, *weight_args)
        else:
            l2r, clr = pl.pallas_call(
                _stage_c_alias,
                grid=(CG,),
                in_specs=(data_specs + weight_specs
                          + [pl.BlockSpec(memory_space=pl.ANY)] * 2),
                out_specs=out_specs,
                out_shape=out_shape,
                input_output_aliases={18: 0, 19: 1},
            )(xh_c, wl2_c, wcl_c, *weight_args, l2r, clr)
    return (l2r.reshape(B, S, TD), clr.reshape(B, S, H))


# speedup vs baseline: 1.0213x; 1.0213x over previous
import functools

import jax
import jax.numpy as jnp
from jax import lax
from jax.experimental import pallas as pl
from jax.experimental.pallas import tpu as pltpu
from jax.experimental.pallas import tpu_sc as plsc

_SQRT_HALF = 0.7071067811865476

_NC, _NS, _L = 2, 16, 16
_NW = _NC * _NS


def _gelu(x):
    return 0.5 * x * (1.0 + jax.lax.erf(x * _SQRT_HALF))


def _dot(a, b):
    return jnp.dot(a, b, preferred_element_type=jnp.float32)


def _bdot(a, b):
    return jnp.dot(a.astype(jnp.bfloat16), b, preferred_element_type=jnp.float32)


def _stage_a(x_ref, Wte_ref, bte_ref, l2e_ref, cle_ref, Wg_ref, bg_ref,
             xh_ref, ltl2_ref, ltcl_ref):
    xh = _gelu(_dot(x_ref[...], Wte_ref[...]) + bte_ref[...])
    xh_ref[...] = xh.astype(jnp.bfloat16)
    gl2 = _dot(xh + l2e_ref[...], Wg_ref[...]) + bg_ref[...]
    gcl = _dot(xh + cle_ref[...], Wg_ref[...]) + bg_ref[...]
    ltl2_ref[...] = gl2.T
    ltcl_ref[...] = gcl.T


def _route_chunk(lt_v, w_v, o, E):
    _BIG = 3.0e38
    vs = [lt_v[e, pl.ds(o, _L)] for e in range(E)]
    m = vs[0]
    for v in vs[1:]:
        m = jnp.maximum(m, v)
    exs = [jnp.exp(v - m) for v in vs]
    z = exs[0]
    for ex in exs[1:]:
        z = z + ex
    zi = 1.0 / z
    s1 = []
    free = None
    for v in vs:
        if free is None:
            s = jnp.where(v == m, 1.0, 0.0)
        else:
            s = jnp.where(v == m, free, 0.0)
        free = (1.0 - s) if free is None else (free - s)
        s1.append(s)
    vs2 = [v - s * _BIG for v, s in zip(vs, s1)]
    m2 = vs2[0]
    for v in vs2[1:]:
        m2 = jnp.maximum(m2, v)
    s2 = []
    free2 = None
    for v in vs2:
        if free2 is None:
            s = jnp.where(v == m2, 1.0, 0.0)
        else:
            s = jnp.where(v == m2, free2, 0.0)
        free2 = (1.0 - s) if free2 is None else (free2 - s)
        s2.append(s)
    for e in range(E):
        w_v[e, pl.ds(o, _L)] = (s1[e] + s2[e]) * (exs[e] * zi)


def _make_router(E, N):
    CH = N // _NW
    mesh = plsc.VectorSubcoreMesh(core_axis_name="c", subcore_axis_name="s")

    @functools.partial(
        pl.kernel,
        out_type=[jax.ShapeDtypeStruct((E, N), jnp.float32),
                  jax.ShapeDtypeStruct((E, N), jnp.float32)],
        mesh=mesh,
        scratch_types=[
            pltpu.VMEM((E, CH), jnp.float32),
            pltpu.VMEM((E, CH), jnp.float32),
            pltpu.VMEM((E, CH), jnp.float32),
            pltpu.VMEM((E, CH), jnp.float32),
        ],
    )
    def _router(ltl2_hbm, ltcl_hbm, wl2_hbm, wcl_hbm, l2_v, cl_v, w1_v, w2_v):
        wid = lax.axis_index("s") * _NC + lax.axis_index("c")
        base = wid * CH
        pltpu.sync_copy(ltl2_hbm.at[:, pl.ds(base, CH)], l2_v)
        pltpu.sync_copy(ltcl_hbm.at[:, pl.ds(base, CH)], cl_v)

        def body(j, carry):
            o = j * _L
            _route_chunk(l2_v, w1_v, o, E)
            _route_chunk(cl_v, w2_v, o, E)
            return carry

        lax.fori_loop(0, CH // _L, body, 0)
        pltpu.sync_copy(w1_v, wl2_hbm.at[:, pl.ds(base, CH)])
        pltpu.sync_copy(w2_v, wcl_hbm.at[:, pl.ds(base, CH)])

    return _router


def _stage_c(xh_ref, wl2t_ref, wclt_ref, l2e_ref, cle_ref, W1c_ref, b1f_ref,
             W2s_ref, b2_ref, rep_ref, lnl2g_ref, lnl2b_ref, lncg_ref,
             lncb_ref, Wt2i_ref, bt2i_ref, Wcl_ref, bcl_ref,
             l2r_ref, clr_ref, *, E, H):
    xh = xh_ref[...].astype(jnp.float32)

    def branch(emb_ref, wt_ref, g_ref, b_ref):
        inp = xh + emb_ref[...]
        w = wt_ref[...].T
        zpre = _bdot(inp, W1c_ref[...]) + b1f_ref[...]
        h = _gelu(zpre.astype(jnp.bfloat16))
        wrep = _bdot(w, rep_ref[...]).astype(jnp.bfloat16)
        moe = _dot(h * wrep, W2s_ref[...]) + _dot(w, b2_ref[...])
        mu = jnp.mean(moe, axis=-1, keepdims=True)
        var = jnp.mean((moe - mu) ** 2, axis=-1, keepdims=True)
        ln = g_ref[...] * (moe - mu) * jax.lax.rsqrt(var + 1e-5) + b_ref[...]
        return _gelu(ln) + inp

    l2o = branch(l2e_ref, wl2t_ref, lnl2g_ref, lnl2b_ref)
    clo = branch(cle_ref, wclt_ref, lncg_ref, lncb_ref)
    l2r_ref[...] = _bdot(l2o, Wt2i_ref[...]) + bt2i_ref[...]
    clr_ref[...] = _bdot(clo, Wcl_ref[...]) + bcl_ref[...]


def kernel(x, Wte, bte, l2_emb, cl_emb, Wg, bg, W1, b1, W2, b2,
           ln_l2_g, ln_l2_b, ln_cl_g, ln_cl_b, Wt2i, bt2i, Wcl, bcl):
    B, S, TD = x.shape
    H = Wte.shape[1]
    E = Wg.shape[1]
    N = B * S
    T = min(1024, N)
    xf = x.reshape(N, TD)
    W1c = W1.transpose(1, 0, 2).reshape(H, E * H).astype(jnp.bfloat16)
    b1f = b1.reshape(1, E * H)
    W2s = W2.reshape(E * H, H).astype(jnp.bfloat16)
    Wt2i = Wt2i.astype(jnp.bfloat16)
    Wcl = Wcl.astype(jnp.bfloat16)
    rep = jnp.repeat(jnp.eye(E, dtype=jnp.bfloat16), H, axis=1)

    row = lambda v: v.reshape(1, -1)
    full = lambda shape: pl.BlockSpec(shape, lambda i: (0, 0))
    grid = (N // T,)

    xh, ltl2, ltcl = pl.pallas_call(
        _stage_a,
        grid=grid,
        in_specs=[
            pl.BlockSpec((T, TD), lambda i: (i, 0)),
            full((TD, H)), full((1, H)), full((1, H)), full((1, H)),
            full((H, E)), full((1, E)),
        ],
        out_specs=[
            pl.BlockSpec((T, H), lambda i: (i, 0)),
            pl.BlockSpec((E, T), lambda i: (0, i)),
            pl.BlockSpec((E, T), lambda i: (0, i)),
        ],
        out_shape=[
            jax.ShapeDtypeStruct((N, H), jnp.bfloat16),
            jax.ShapeDtypeStruct((E, N), jnp.float32),
            jax.ShapeDtypeStruct((E, N), jnp.float32),
        ],
    )(xf, Wte, row(bte), row(l2_emb), row(cl_emb), Wg, row(bg))

    wl2t, wclt = _make_router(E, N)(ltl2, ltcl)

    out = pl.pallas_call(
        functools.partial(_stage_c, E=E, H=H),
        grid=grid,
        in_specs=[
            pl.BlockSpec((T, H), lambda i: (i, 0)),
            pl.BlockSpec((E, T), lambda i: (0, i)),
            pl.BlockSpec((E, T), lambda i: (0, i)),
            full((1, H)), full((1, H)),
            full((H, E * H)), full((1, E * H)),
            full((E * H, H)), full((E, H)), full((E, E * H)),
            full((1, H)), full((1, H)), full((1, H)), full((1, H)),
            full((H, TD)), full((1, TD)),
            full((H, H)), full((1, H)),
        ],
        out_specs=[
            pl.BlockSpec((T, TD), lambda i: (i, 0)),
            pl.BlockSpec((T, H), lambda i: (i, 0)),
        ],
        out_shape=[
            jax.ShapeDtypeStruct((N, TD), jnp.float32),
            jax.ShapeDtypeStruct((N, H), jnp.float32),
        ],
    )(xh, wl2t, wclt, row(l2_emb), row(cl_emb), W1c, b1f, W2s, b2, rep,
      row(ln_l2_g), row(ln_l2_b), row(ln_cl_g), row(ln_cl_b),
      Wt2i, row(bt2i), Wcl, row(bcl))
    l2r, clr = out
    return (l2r.reshape(B, S, TD), clr.reshape(B, S, H))
